# baseline (speedup 1.0000x reference)
import functools

import jax
import jax.numpy as jnp
from jax import lax
from jax.experimental import pallas as pl
from jax.experimental.pallas import tpu as pltpu
from jax.experimental.pallas import tpu_sc as plsc


@functools.cache
def _make_sc_gather(V, D, B):
    info = plsc.get_sparse_core_info()
    NC, NS = info.num_cores, info.num_subcores
    NW = NC * NS
    assert B % NW == 0 and B % (8 * NW) == 0 and D % info.num_lanes == 0
    b_per_w = B // NW
    mesh = plsc.VectorSubcoreMesh(core_axis_name="c", subcore_axis_name="s")

    @functools.partial(
        pl.kernel, mesh=mesh,
        out_type=jax.ShapeDtypeStruct((B, D), jnp.float32),
        scratch_types=[
            pltpu.VMEM((b_per_w,), jnp.int32),
            pltpu.VMEM((b_per_w, D), jnp.float32),
            pltpu.SemaphoreType.DMA,
        ],
        compiler_params=pltpu.CompilerParams(use_tc_tiling_on_sc=False),
    )
    def gather_k(table_hbm, idx_hbm, out_hbm, idx_v, rows_v, sem):
        wid = lax.axis_index("s") * NC + lax.axis_index("c")
        base = wid * b_per_w
        pltpu.sync_copy(idx_hbm.at[pl.ds(base, b_per_w)], idx_v)
        pltpu.async_copy(table_hbm.at[idx_v], rows_v, sem).wait()
        pltpu.sync_copy(rows_v, out_hbm.at[pl.ds(base, b_per_w)])

    return gather_k


def _mlp_body(e_ref, W1_ref, b1_ref, W2_ref, b2_ref, W3_ref, b3_ref,
              out_ref, h2_ref):
    @pl.when(pl.program_id(0) == 0)
    def _():
        h1 = jnp.dot(e_ref[...], W1_ref[...],
                     preferred_element_type=jnp.float32) + b1_ref[...]
        h1 = jnp.maximum(h1, 0.0)
        h2 = jnp.dot(h1, W2_ref[...],
                     preferred_element_type=jnp.float32) + b2_ref[...]
        h2_ref[...] = jnp.maximum(h2, 0.0)

    out_ref[...] = jnp.dot(h2_ref[...], W3_ref[...],
                           preferred_element_type=jnp.float32) + b3_ref[...]


@functools.partial(jax.jit, static_argnames=("tile_n",))
def _mlp(e, W1, b1, W2, b2, W3, b3, tile_n=2048):
    B, D = e.shape
    H1 = W1.shape[1]
    H2 = W2.shape[1]
    N = W3.shape[1]
    grid = (pl.cdiv(N, tile_n),)
    return pl.pallas_call(
        _mlp_body,
        grid=grid,
        in_specs=[
            pl.BlockSpec((B, D), lambda i: (0, 0)),
            pl.BlockSpec((D, H1), lambda i: (0, 0)),
            pl.BlockSpec((1, H1), lambda i: (0, 0)),
            pl.BlockSpec((H1, H2), lambda i: (0, 0)),
            pl.BlockSpec((1, H2), lambda i: (0, 0)),
            pl.BlockSpec((D, tile_n), lambda i: (0, i)),
            pl.BlockSpec((1, tile_n), lambda i: (0, i)),
        ],
        out_specs=pl.BlockSpec((B, tile_n), lambda i: (0, i)),
        out_shape=jax.ShapeDtypeStruct((B, N), jnp.float32),
        scratch_shapes=[pltpu.VMEM((B, H2), jnp.float32)],
        compiler_params=pltpu.CompilerParams(
            dimension_semantics=("arbitrary",),
        ),
    )(e, W1, b1.reshape(1, H1), W2, b2.reshape(1, H2), W3, b3.reshape(1, N))


def kernel(user_ids, table, W1, b1, W2, b2, W3, b3):
    V, D = table.shape
    B = user_ids.shape[0]
    e = _make_sc_gather(V, D, B)(table, user_ids.astype(jnp.int32))
    return _mlp(e, W1, b1, W2, b2, W3, b3)


# speedup vs baseline: 1.3228x; 1.3228x over previous
import functools

import jax
import jax.numpy as jnp
from jax import lax
from jax.experimental import pallas as pl
from jax.experimental.pallas import tpu as pltpu

_UNROLL = 8


def _body(ids_ref, table_ref, W1_ref, b1_ref, W2_ref, b2_ref, W3_ref, b3_ref,
          out_ref, e_ref, h2_ref, sem):
    B = e_ref.shape[0]

    @pl.when(pl.program_id(0) == 0)
    def _():
        def issue(r, c):
            for j in range(_UNROLL):
                i = r * _UNROLL + j
                row = ids_ref[i]
                pltpu.make_async_copy(
                    table_ref.at[pl.ds(row, 1), :],
                    e_ref.at[pl.ds(i, 1), :],
                    sem,
                ).start()
            return c

        lax.fori_loop(0, B // _UNROLL, issue, 0)
        pltpu.make_async_copy(table_ref.at[pl.ds(0, B), :], e_ref, sem).wait()

        h1 = jnp.dot(e_ref[...], W1_ref[...],
                     preferred_element_type=jnp.float32) + b1_ref[...]
        h1 = jnp.maximum(h1, 0.0)
        h2 = jnp.dot(h1, W2_ref[...],
                     preferred_element_type=jnp.float32) + b2_ref[...]
        h2_ref[...] = jnp.maximum(h2, 0.0).astype(jnp.bfloat16)

    w3 = W3_ref[...].astype(jnp.bfloat16)
    out_ref[...] = jnp.dot(h2_ref[...], w3,
                           preferred_element_type=jnp.float32) + b3_ref[...]


@functools.partial(jax.jit, static_argnames=("tile_n",))
def _fused(user_ids, table, W1, b1, W2, b2, W3, b3, tile_n=2048):
    B = user_ids.shape[0]
    D = table.shape[1]
    H1 = W1.shape[1]
    H2 = W2.shape[1]
    N = W3.shape[1]
    grid = (pl.cdiv(N, tile_n),)
    grid_spec = pltpu.PrefetchScalarGridSpec(
        num_scalar_prefetch=1,
        grid=grid,
        in_specs=[
            pl.BlockSpec(memory_space=pltpu.HBM),
            pl.BlockSpec((D, H1), lambda i, ids: (0, 0)),
            pl.BlockSpec((1, H1), lambda i, ids: (0, 0)),
            pl.BlockSpec((H1, H2), lambda i, ids: (0, 0)),
            pl.BlockSpec((1, H2), lambda i, ids: (0, 0)),
            pl.BlockSpec((D, tile_n), lambda i, ids: (0, i)),
            pl.BlockSpec((1, tile_n), lambda i, ids: (0, i)),
        ],
        out_specs=pl.BlockSpec((B, tile_n), lambda i, ids: (0, i)),
        scratch_shapes=[
            pltpu.VMEM((B, D), jnp.float32),
            pltpu.VMEM((B, H2), jnp.bfloat16),
            pltpu.SemaphoreType.DMA,
        ],
    )
    return pl.pallas_call(
        _body,
        grid_spec=grid_spec,
        out_shape=jax.ShapeDtypeStruct((B, N), jnp.float32),
        compiler_params=pltpu.CompilerParams(
            dimension_semantics=("arbitrary",),
        ),
    )(user_ids.astype(jnp.int32), table, W1, b1.reshape(1, H1), W2,
      b2.reshape(1, H2), W3, b3.reshape(1, N))


def kernel(user_ids, table, W1, b1, W2, b2, W3, b3):
    return _fused(user_ids, table, W1, b1, W2, b2, W3, b3)
